# baseline (speedup 1.0000x reference)
import jax
import jax.numpy as jnp
from jax.experimental import pallas as pl
from jax.experimental.pallas import tpu as pltpu

N = 100000
D = 128
K = 16
OUT = 256
TN = 5000


def _pass1(x_ref, ev_ref, evals_ref, mask_ref, w1t_ref, b1_ref, w2t_ref,
           b2_ref, wp_ref, m_ref, acc_ref):
    i = pl.program_id(0)

    @pl.when(i == 0)
    def _():
        acc_ref[...] = jnp.zeros_like(acc_ref)

    acc_ref[...] += jax.lax.dot_general(
        ev_ref[...], x_ref[...],
        dimension_numbers=(((0,), (0,)), ((), ())),
        preferred_element_type=jnp.float32)

    @pl.when(i == pl.num_programs(0) - 1)
    def _():
        h = jnp.maximum(
            jnp.dot(w1t_ref[...], evals_ref[...],
                    preferred_element_type=jnp.float32) + b1_ref[...], 0.0)
        filt = jnp.tanh(
            jnp.dot(w2t_ref[...], h,
                    preferred_element_type=jnp.float32) + b2_ref[...])
        filt = filt * mask_ref[...]
        m_ref[...] = jnp.dot(filt * acc_ref[...], wp_ref[...],
                             preferred_element_type=jnp.float32)


def _pass2(ev_ref, m_ref, bp_ref, out_ref):
    out_ref[...] = jnp.dot(ev_ref[...], m_ref[...],
                           preferred_element_type=jnp.float32) + bp_ref[...]


def kernel(x, eigvecs, eigvals, eig_mask, W1, b1, W2, b2, Wp, bp):
    ntiles = N // TN
    evals_col = eigvals.reshape(K, 1)
    mask_col = eig_mask.astype(jnp.float32).reshape(K, 1)
    w1t = W1.T
    b1_col = b1.reshape(K // 2, 1)
    w2t = W2.T
    b2_col = b2.reshape(K, 1)
    bp_row = bp.reshape(1, OUT)

    m = pl.pallas_call(
        _pass1,
        grid=(ntiles,),
        in_specs=[
            pl.BlockSpec((TN, D), lambda i: (i, 0)),
            pl.BlockSpec((TN, K), lambda i: (i, 0)),
            pl.BlockSpec((K, 1), lambda i: (0, 0)),
            pl.BlockSpec((K, 1), lambda i: (0, 0)),
            pl.BlockSpec((K // 2, K), lambda i: (0, 0)),
            pl.BlockSpec((K // 2, 1), lambda i: (0, 0)),
            pl.BlockSpec((K, K // 2), lambda i: (0, 0)),
            pl.BlockSpec((K, 1), lambda i: (0, 0)),
            pl.BlockSpec((D, OUT), lambda i: (0, 0)),
        ],
        out_specs=pl.BlockSpec((K, OUT), lambda i: (0, 0)),
        out_shape=jax.ShapeDtypeStruct((K, OUT), jnp.float32),
        scratch_shapes=[pltpu.VMEM((K, D), jnp.float32)],
    )(x, eigvecs, evals_col, mask_col, w1t, b1_col, w2t, b2_col, Wp)

    out = pl.pallas_call(
        _pass2,
        grid=(ntiles,),
        in_specs=[
            pl.BlockSpec((TN, K), lambda i: (i, 0)),
            pl.BlockSpec((K, OUT), lambda i: (0, 0)),
            pl.BlockSpec((1, OUT), lambda i: (0, 0)),
        ],
        out_specs=pl.BlockSpec((TN, OUT), lambda i: (i, 0)),
        out_shape=jax.ShapeDtypeStruct((N, OUT), jnp.float32),
    )(eigvecs, m, bp_row)
    return out
